# baseline (speedup 1.0000x reference)
import jax
import jax.numpy as jnp
from jax import lax
from jax.experimental import pallas as pl
from jax.experimental.pallas import tpu as pltpu
from jax.experimental.pallas import tpu_sc as plsc

N_NODES = 10000
N_EDGES = 320000
D_FEAT = 128
IN_CHANNELS = 512
OUT_CHANNELS = 64

NC = 2
NS = 16
NW = NC * NS

HIST_PAD = 10240
EPW = N_EDGES // NW
EGROUPS = EPW // 16

STRIPE = 320
LAST_BASE = N_NODES - STRIPE

_mesh = plsc.VectorSubcoreMesh(core_axis_name="c", subcore_axis_name="s")


ZSTRIPE = HIST_PAD // NS


def _hist_body(edge_ref, ones_ref, x_ref, p0_ref, p1_ref, out_ref,
               idx_v, ones_v, zeros_v, xb, hist_sh, sem_e, sem_o, sem_x,
               sem_w):
    c = lax.axis_index("c")
    s = lax.axis_index("s")
    w = s * NC + c
    base = jnp.minimum(w * STRIPE, LAST_BASE)

    cp_e = pltpu.make_async_copy(
        edge_ref.at[pl.ds(N_EDGES + w * EPW, EPW)], idx_v, sem_e)
    cp_e.start()
    cp_o = pltpu.make_async_copy(ones_ref, ones_v, sem_o)
    cp_o.start()
    cp_x = pltpu.make_async_copy(x_ref.at[pl.ds(base, STRIPE)], xb, sem_x)
    cp_x.start()

    def fill_zeros(i, carry):
        zeros_v[pl.ds(i * 16, 16)] = jnp.zeros((16,), jnp.int32)
        return carry

    lax.fori_loop(0, ZSTRIPE // 16, fill_zeros, 0)

    pltpu.sync_copy(zeros_v, hist_sh.at[pl.ds(s * ZSTRIPE, ZSTRIPE)])

    cp_x.wait()
    cp_w = pltpu.make_async_copy(
        xb, out_ref.at[pl.ds(base, STRIPE), pl.ds(0, D_FEAT)], sem_w)
    cp_w.start()

    cp_e.wait()
    cp_o.wait()
    plsc.subcore_barrier()
    pltpu.sync_copy(ones_v, hist_sh.at[idx_v], add=True)
    plsc.subcore_barrier()

    @pl.when(c == 0)
    def _():
        pltpu.sync_copy(hist_sh.at[pl.ds(s * ZSTRIPE, ZSTRIPE)],
                        p0_ref.at[pl.ds(s * ZSTRIPE, ZSTRIPE)])

    @pl.when(c == 1)
    def _():
        pltpu.sync_copy(hist_sh.at[pl.ds(s * ZSTRIPE, ZSTRIPE)],
                        p1_ref.at[pl.ds(s * ZSTRIPE, ZSTRIPE)])

    cp_w.wait()


_hist_kernel = pl.kernel(
    _hist_body,
    out_type=(jax.ShapeDtypeStruct((HIST_PAD,), jnp.int32),
              jax.ShapeDtypeStruct((HIST_PAD,), jnp.int32),
              jax.ShapeDtypeStruct((N_NODES, D_FEAT + OUT_CHANNELS),
                                   jnp.float32)),
    mesh=_mesh,
    scratch_types=[
        pltpu.VMEM((EPW,), jnp.int32),
        pltpu.VMEM((EPW,), jnp.int32),
        pltpu.VMEM((ZSTRIPE,), jnp.int32),
        pltpu.VMEM((STRIPE, D_FEAT), jnp.float32),
        pltpu.VMEM_SHARED((HIST_PAD,), jnp.int32),
        pltpu.SemaphoreType.DMA,
        pltpu.SemaphoreType.DMA,
        pltpu.SemaphoreType.DMA,
        pltpu.SemaphoreType.DMA,
    ],
)


_BLK = 1024
_GRID = (N_NODES + _BLK - 1) // _BLK
_SUB = _BLK // 128


_LAST_ROWS = N_NODES - (_GRID - 1) * _BLK


def _lookup_body(p0_ref, p1_ref, emb_ref, stage_ref, out_ref, ev, sem):
    del stage_ref
    i = pl.program_id(0)
    emb = emb_ref[...].astype(jnp.bfloat16)
    iota_k = lax.broadcasted_iota(jnp.int16, (IN_CHANNELS, 128), 0)
    for r in range(_SUB):
        deg_row = jnp.minimum(p0_ref[r:r + 1, :] + p1_ref[r:r + 1, :],
                              jnp.int32(IN_CHANNELS - 1)).astype(jnp.int16)
        pt = (jnp.broadcast_to(deg_row, (IN_CHANNELS, 128)) == iota_k)
        pt = pt.astype(jnp.bfloat16)
        e = lax.dot_general(pt, emb, (((0,), (0,)), ((), ())),
                            preferred_element_type=jnp.float32)
        ev[pl.ds(r * 128, 128), :] = e

    @pl.when(i < _GRID - 1)
    def _():
        cp = pltpu.make_async_copy(
            ev, out_ref.at[pl.ds(i * _BLK, _BLK),
                           pl.ds(D_FEAT, OUT_CHANNELS)], sem)
        cp.start()
        cp.wait()

    @pl.when(i == _GRID - 1)
    def _():
        cp = pltpu.make_async_copy(
            ev.at[pl.ds(0, _LAST_ROWS)],
            out_ref.at[pl.ds((_GRID - 1) * _BLK, _LAST_ROWS),
                       pl.ds(D_FEAT, OUT_CHANNELS)], sem)
        cp.start()
        cp.wait()


_lookup_kernel = pl.pallas_call(
    _lookup_body,
    grid=(_GRID,),
    in_specs=[
        pl.BlockSpec((_SUB, 128), lambda i: (i, 0)),
        pl.BlockSpec((_SUB, 128), lambda i: (i, 0)),
        pl.BlockSpec((IN_CHANNELS, OUT_CHANNELS), lambda i: (0, 0)),
        pl.BlockSpec(memory_space=pl.ANY),
    ],
    out_specs=pl.BlockSpec(memory_space=pl.ANY),
    out_shape=jax.ShapeDtypeStruct((N_NODES, D_FEAT + OUT_CHANNELS),
                                   jnp.float32),
    input_output_aliases={3: 0},
    scratch_shapes=[
        pltpu.VMEM((_BLK, OUT_CHANNELS), jnp.float32),
        pltpu.SemaphoreType.DMA,
    ],
)


def kernel(x, edge_index, emb_table):
    edge_flat = edge_index.astype(jnp.int32).reshape(2 * N_EDGES)
    ones = jnp.full((EPW,), 1, jnp.int32)
    p0, p1, staged = _hist_kernel(edge_flat, ones, x)
    p0v = p0.reshape(HIST_PAD // 128, 128)
    p1v = p1.reshape(HIST_PAD // 128, 128)
    return _lookup_kernel(p0v, p1v, emb_table, staged)


# speedup vs baseline: 1.0641x; 1.0641x over previous
import jax
import jax.numpy as jnp
from jax import lax
from jax.experimental import pallas as pl
from jax.experimental.pallas import tpu as pltpu
from jax.experimental.pallas import tpu_sc as plsc

N_NODES = 10000
N_EDGES = 320000
D_FEAT = 128
IN_CHANNELS = 512
OUT_CHANNELS = 64

NC = 2
NS = 16
NW = NC * NS

HIST_PAD = 10240
EPW = N_EDGES // NW
EGROUPS = EPW // 16

STRIPE = 320
LAST_BASE = N_NODES - STRIPE

_mesh = plsc.VectorSubcoreMesh(core_axis_name="c", subcore_axis_name="s")


ZSTRIPE = HIST_PAD // NS


def _hist_body(edge_ref, ones_ref, x_ref, p0_ref, p1_ref, out_ref,
               idx_v, ones_v, zeros_v, xb, hist_sh, sem_e, sem_o, sem_x,
               sem_w):
    c = lax.axis_index("c")
    s = lax.axis_index("s")
    w = s * NC + c
    base = jnp.minimum(w * STRIPE, LAST_BASE)

    cp_e = pltpu.make_async_copy(
        edge_ref.at[pl.ds(N_EDGES + w * EPW, EPW)], idx_v, sem_e)
    cp_e.start()
    cp_o = pltpu.make_async_copy(ones_ref, ones_v, sem_o)
    cp_o.start()
    cp_x = pltpu.make_async_copy(x_ref.at[pl.ds(base, STRIPE)], xb, sem_x)
    cp_x.start()

    def fill_zeros(i, carry):
        zeros_v[pl.ds(i * 16, 16)] = jnp.zeros((16,), jnp.int32)
        return carry

    lax.fori_loop(0, ZSTRIPE // 16, fill_zeros, 0)

    pltpu.sync_copy(zeros_v, hist_sh.at[pl.ds(s * ZSTRIPE, ZSTRIPE)])

    cp_x.wait()
    cp_w = pltpu.make_async_copy(
        xb, out_ref.at[pl.ds(base, STRIPE), pl.ds(0, D_FEAT)], sem_w)
    cp_w.start()

    cp_e.wait()
    cp_o.wait()
    plsc.subcore_barrier()
    pltpu.sync_copy(ones_v, hist_sh.at[idx_v], add=True)
    plsc.subcore_barrier()

    @pl.when(c == 0)
    def _():
        pltpu.sync_copy(hist_sh.at[pl.ds(s * ZSTRIPE, ZSTRIPE)],
                        p0_ref.at[pl.ds(s * ZSTRIPE, ZSTRIPE)])

    @pl.when(c == 1)
    def _():
        pltpu.sync_copy(hist_sh.at[pl.ds(s * ZSTRIPE, ZSTRIPE)],
                        p1_ref.at[pl.ds(s * ZSTRIPE, ZSTRIPE)])

    cp_w.wait()


_hist_kernel = pl.kernel(
    _hist_body,
    out_type=(jax.ShapeDtypeStruct((HIST_PAD,), jnp.int32),
              jax.ShapeDtypeStruct((HIST_PAD,), jnp.int32),
              jax.ShapeDtypeStruct((N_NODES, D_FEAT + OUT_CHANNELS),
                                   jnp.float32)),
    mesh=_mesh,
    scratch_types=[
        pltpu.VMEM((EPW,), jnp.int32),
        pltpu.VMEM((EPW,), jnp.int32),
        pltpu.VMEM((ZSTRIPE,), jnp.int32),
        pltpu.VMEM((STRIPE, D_FEAT), jnp.float32),
        pltpu.VMEM_SHARED((HIST_PAD,), jnp.int32),
        pltpu.SemaphoreType.DMA,
        pltpu.SemaphoreType.DMA,
        pltpu.SemaphoreType.DMA,
        pltpu.SemaphoreType.DMA,
    ],
)


_BLK = 1024
_GRID = (N_NODES + _BLK - 1) // _BLK
_SUB = _BLK // 128


_LAST_ROWS = N_NODES - (_GRID - 1) * _BLK


def _lookup_body(p0_ref, p1_ref, emb_ref, stage_ref, out_ref, ev, sem):
    del stage_ref
    i = pl.program_id(0)
    emb = emb_ref[...].astype(jnp.bfloat16)
    deg = jnp.minimum(p0_ref[...] + p1_ref[...],
                      jnp.int32(IN_CHANNELS - 1))
    iota_k = lax.broadcasted_iota(jnp.int32, (IN_CHANNELS, 2 * 128), 0)
    for r in range(_SUB // 2):
        two = deg[2 * r:2 * r + 2, :].reshape(1, 2 * 128)
        pt = (jnp.broadcast_to(two, (IN_CHANNELS, 2 * 128)) == iota_k)
        pt = pt.astype(jnp.bfloat16)
        e = lax.dot_general(pt, emb, (((0,), (0,)), ((), ())),
                            preferred_element_type=jnp.float32)
        ev[pl.ds(r * 256, 256), :] = e

    @pl.when(i < _GRID - 1)
    def _():
        cp = pltpu.make_async_copy(
            ev, out_ref.at[pl.ds(i * _BLK, _BLK),
                           pl.ds(D_FEAT, OUT_CHANNELS)], sem)
        cp.start()
        cp.wait()

    @pl.when(i == _GRID - 1)
    def _():
        cp = pltpu.make_async_copy(
            ev.at[pl.ds(0, _LAST_ROWS)],
            out_ref.at[pl.ds((_GRID - 1) * _BLK, _LAST_ROWS),
                       pl.ds(D_FEAT, OUT_CHANNELS)], sem)
        cp.start()
        cp.wait()


_lookup_kernel = pl.pallas_call(
    _lookup_body,
    grid=(_GRID,),
    in_specs=[
        pl.BlockSpec((_SUB, 128), lambda i: (i, 0)),
        pl.BlockSpec((_SUB, 128), lambda i: (i, 0)),
        pl.BlockSpec((IN_CHANNELS, OUT_CHANNELS), lambda i: (0, 0)),
        pl.BlockSpec(memory_space=pl.ANY),
    ],
    out_specs=pl.BlockSpec(memory_space=pl.ANY),
    out_shape=jax.ShapeDtypeStruct((N_NODES, D_FEAT + OUT_CHANNELS),
                                   jnp.float32),
    input_output_aliases={3: 0},
    scratch_shapes=[
        pltpu.VMEM((_BLK, OUT_CHANNELS), jnp.float32),
        pltpu.SemaphoreType.DMA,
    ],
)


def kernel(x, edge_index, emb_table):
    edge_flat = edge_index.astype(jnp.int32).reshape(2 * N_EDGES)
    ones = jnp.full((EPW,), 1, jnp.int32)
    p0, p1, staged = _hist_kernel(edge_flat, ones, x)
    p0v = p0.reshape(HIST_PAD // 128, 128)
    p1v = p1.reshape(HIST_PAD // 128, 128)
    return _lookup_kernel(p0v, p1v, emb_table, staged)


# speedup vs baseline: 1.1670x; 1.0967x over previous
import jax
import jax.numpy as jnp
from jax import lax
from jax.experimental import pallas as pl
from jax.experimental.pallas import tpu as pltpu
from jax.experimental.pallas import tpu_sc as plsc

N_NODES = 10000
N_EDGES = 320000
D_FEAT = 128
IN_CHANNELS = 512
OUT_CHANNELS = 64

NC = 2
NS = 16
NW = NC * NS

HIST_PAD = 10240
EPW = N_EDGES // NW
ZSTRIPE = HIST_PAD // NS

_mesh = plsc.VectorSubcoreMesh(core_axis_name="c", subcore_axis_name="s")


def _hist_body(edge_ref, ones_ref, p0_ref, p1_ref,
               idx_v, ones_v, zeros_v, hist_sh, sem_e, sem_o):
    c = lax.axis_index("c")
    s = lax.axis_index("s")
    w = s * NC + c

    cp_e = pltpu.make_async_copy(
        edge_ref.at[pl.ds(N_EDGES + w * EPW, EPW)], idx_v, sem_e)
    cp_e.start()
    cp_o = pltpu.make_async_copy(ones_ref, ones_v, sem_o)
    cp_o.start()

    def fill_zeros(i, carry):
        zeros_v[pl.ds(i * 16, 16)] = jnp.zeros((16,), jnp.int32)
        return carry

    lax.fori_loop(0, ZSTRIPE // 16, fill_zeros, 0)

    pltpu.sync_copy(zeros_v, hist_sh.at[pl.ds(s * ZSTRIPE, ZSTRIPE)])

    cp_e.wait()
    cp_o.wait()
    plsc.subcore_barrier()
    pltpu.sync_copy(ones_v, hist_sh.at[idx_v], add=True)
    plsc.subcore_barrier()

    @pl.when(c == 0)
    def _():
        pltpu.sync_copy(hist_sh.at[pl.ds(s * ZSTRIPE, ZSTRIPE)],
                        p0_ref.at[pl.ds(s * ZSTRIPE, ZSTRIPE)])

    @pl.when(c == 1)
    def _():
        pltpu.sync_copy(hist_sh.at[pl.ds(s * ZSTRIPE, ZSTRIPE)],
                        p1_ref.at[pl.ds(s * ZSTRIPE, ZSTRIPE)])


_hist_kernel = pl.kernel(
    _hist_body,
    out_type=(jax.ShapeDtypeStruct((HIST_PAD,), jnp.int32),
              jax.ShapeDtypeStruct((HIST_PAD,), jnp.int32)),
    mesh=_mesh,
    scratch_types=[
        pltpu.VMEM((EPW,), jnp.int32),
        pltpu.VMEM((EPW,), jnp.int32),
        pltpu.VMEM((ZSTRIPE,), jnp.int32),
        pltpu.VMEM_SHARED((HIST_PAD,), jnp.int32),
        pltpu.SemaphoreType.DMA,
        pltpu.SemaphoreType.DMA,
    ],
)


_BLK = 1024
_GRID = (N_NODES + _BLK - 1) // _BLK
_SUB = _BLK // 128


def _lookup_body(p0_ref, p1_ref, x_ref, emb_ref, out_ref):
    out_ref[:, 0:D_FEAT] = x_ref[...]
    emb = emb_ref[...].astype(jnp.bfloat16)
    deg = jnp.minimum(p0_ref[...] + p1_ref[...],
                      jnp.int32(IN_CHANNELS - 1))
    iota_k = lax.broadcasted_iota(jnp.int32, (IN_CHANNELS, 2 * 128), 0)
    for r in range(_SUB // 2):
        two = deg[2 * r:2 * r + 2, :].reshape(1, 2 * 128)
        pt = (jnp.broadcast_to(two, (IN_CHANNELS, 2 * 128)) == iota_k)
        pt = pt.astype(jnp.bfloat16)
        e = lax.dot_general(pt, emb, (((0,), (0,)), ((), ())),
                            preferred_element_type=jnp.float32)
        out_ref[pl.ds(r * 256, 256), pl.ds(D_FEAT, OUT_CHANNELS)] = e


_lookup_kernel = pl.pallas_call(
    _lookup_body,
    grid=(_GRID,),
    in_specs=[
        pl.BlockSpec((_SUB, 128), lambda i: (i, 0)),
        pl.BlockSpec((_SUB, 128), lambda i: (i, 0)),
        pl.BlockSpec((_BLK, D_FEAT), lambda i: (i, 0)),
        pl.BlockSpec((IN_CHANNELS, OUT_CHANNELS), lambda i: (0, 0)),
    ],
    out_specs=pl.BlockSpec((_BLK, D_FEAT + OUT_CHANNELS), lambda i: (i, 0)),
    out_shape=jax.ShapeDtypeStruct((N_NODES, D_FEAT + OUT_CHANNELS),
                                   jnp.float32),
)


def kernel(x, edge_index, emb_table):
    edge_flat = edge_index.astype(jnp.int32).reshape(2 * N_EDGES)
    ones = jnp.full((EPW,), 1, jnp.int32)
    p0, p1 = _hist_kernel(edge_flat, ones)
    p0v = p0.reshape(HIST_PAD // 128, 128)
    p1v = p1.reshape(HIST_PAD // 128, 128)
    return _lookup_kernel(p0v, p1v, x, emb_table)


# speedup vs baseline: 1.1687x; 1.0014x over previous
import jax
import jax.numpy as jnp
from jax import lax
from jax.experimental import pallas as pl
from jax.experimental.pallas import tpu as pltpu
from jax.experimental.pallas import tpu_sc as plsc

N_NODES = 10000
N_EDGES = 320000
D_FEAT = 128
IN_CHANNELS = 512
OUT_CHANNELS = 64

NC = 2
NS = 16
NW = NC * NS

HIST_PAD = 10240
EPW = N_EDGES // NW
ZSTRIPE = HIST_PAD // NS

_mesh = plsc.VectorSubcoreMesh(core_axis_name="c", subcore_axis_name="s")


def _hist_body(edge_ref, ones_ref, p0_ref, p1_ref,
               idx_v, ones_v, zeros_v, hist_sh, sem_e, sem_o):
    c = lax.axis_index("c")
    s = lax.axis_index("s")
    w = s * NC + c

    cp_e = pltpu.make_async_copy(
        edge_ref.at[pl.ds(N_EDGES + w * EPW, EPW)], idx_v, sem_e)
    cp_e.start()
    cp_o = pltpu.make_async_copy(ones_ref, ones_v, sem_o)
    cp_o.start()

    def fill_zeros(i, carry):
        zeros_v[pl.ds(i * 16, 16)] = jnp.zeros((16,), jnp.int32)
        return carry

    lax.fori_loop(0, ZSTRIPE // 16, fill_zeros, 0)

    pltpu.sync_copy(zeros_v, hist_sh.at[pl.ds(s * ZSTRIPE, ZSTRIPE)])

    cp_e.wait()
    cp_o.wait()
    plsc.subcore_barrier()
    pltpu.sync_copy(ones_v, hist_sh.at[idx_v], add=True)
    plsc.subcore_barrier()

    @pl.when(c == 0)
    def _():
        pltpu.sync_copy(hist_sh.at[pl.ds(s * ZSTRIPE, ZSTRIPE)],
                        p0_ref.at[pl.ds(s * ZSTRIPE, ZSTRIPE)])

    @pl.when(c == 1)
    def _():
        pltpu.sync_copy(hist_sh.at[pl.ds(s * ZSTRIPE, ZSTRIPE)],
                        p1_ref.at[pl.ds(s * ZSTRIPE, ZSTRIPE)])


_hist_kernel = pl.kernel(
    _hist_body,
    out_type=(jax.ShapeDtypeStruct((HIST_PAD,), jnp.int32),
              jax.ShapeDtypeStruct((HIST_PAD,), jnp.int32)),
    mesh=_mesh,
    scratch_types=[
        pltpu.VMEM((EPW,), jnp.int32),
        pltpu.VMEM((EPW,), jnp.int32),
        pltpu.VMEM((ZSTRIPE,), jnp.int32),
        pltpu.VMEM_SHARED((HIST_PAD,), jnp.int32),
        pltpu.SemaphoreType.DMA,
        pltpu.SemaphoreType.DMA,
    ],
)


_BLK = 1024
_GRID = (N_NODES + _BLK - 1) // _BLK
_SUB = _BLK // 128


def _lookup_body(p0_ref, p1_ref, x_ref, emb_ref, out_ref):
    out_ref[:, 0:D_FEAT] = x_ref[...]
    emb = emb_ref[...].astype(jnp.bfloat16)
    deg = jnp.minimum(p0_ref[...] + p1_ref[...],
                      jnp.int32(IN_CHANNELS - 1))
    iota_k = lax.broadcasted_iota(jnp.int32, (IN_CHANNELS, 128), 0)
    for r in range(_SUB):
        deg_row = deg[r:r + 1, :]
        pt = (jnp.broadcast_to(deg_row, (IN_CHANNELS, 128)) == iota_k)
        pt = pt.astype(jnp.bfloat16)
        e = lax.dot_general(pt, emb, (((0,), (0,)), ((), ())),
                            preferred_element_type=jnp.float32)
        out_ref[pl.ds(r * 128, 128), pl.ds(D_FEAT, OUT_CHANNELS)] = e


_lookup_kernel = pl.pallas_call(
    _lookup_body,
    grid=(_GRID,),
    in_specs=[
        pl.BlockSpec((_SUB, 128), lambda i: (i, 0)),
        pl.BlockSpec((_SUB, 128), lambda i: (i, 0)),
        pl.BlockSpec((_BLK, D_FEAT), lambda i: (i, 0)),
        pl.BlockSpec((IN_CHANNELS, OUT_CHANNELS), lambda i: (0, 0)),
    ],
    out_specs=pl.BlockSpec((_BLK, D_FEAT + OUT_CHANNELS), lambda i: (i, 0)),
    out_shape=jax.ShapeDtypeStruct((N_NODES, D_FEAT + OUT_CHANNELS),
                                   jnp.float32),
)


def kernel(x, edge_index, emb_table):
    edge_flat = edge_index.astype(jnp.int32).reshape(2 * N_EDGES)
    ones = jnp.full((EPW,), 1, jnp.int32)
    p0, p1 = _hist_kernel(edge_flat, ones)
    p0v = p0.reshape(HIST_PAD // 128, 128)
    p1v = p1.reshape(HIST_PAD // 128, 128)
    return _lookup_kernel(p0v, p1v, x, emb_table)


# speedup vs baseline: 1.2172x; 1.0415x over previous
import jax
import jax.numpy as jnp
from jax import lax
from jax.experimental import pallas as pl
from jax.experimental.pallas import tpu as pltpu
from jax.experimental.pallas import tpu_sc as plsc

N_NODES = 10000
N_EDGES = 320000
D_FEAT = 128
IN_CHANNELS = 512
OUT_CHANNELS = 64

NC = 2
NS = 16
NW = NC * NS

HIST_PAD = 10240
EPW = N_EDGES // NW
ZSTRIPE = HIST_PAD // NS

_mesh = plsc.VectorSubcoreMesh(core_axis_name="c", subcore_axis_name="s")


def _hist_body(edge_ref, ones_ref, p0_ref, p1_ref,
               idx_v, ones_v, zeros_v, hist_sh, sem_e, sem_o):
    c = lax.axis_index("c")
    s = lax.axis_index("s")
    w = s * NC + c

    cp_e = pltpu.make_async_copy(
        edge_ref.at[pl.ds(N_EDGES + w * EPW, EPW)], idx_v, sem_e)
    cp_e.start()
    cp_o = pltpu.make_async_copy(ones_ref, ones_v, sem_o)
    cp_o.start()

    def fill_zeros(i, carry):
        zeros_v[pl.ds(i * 16, 16)] = jnp.zeros((16,), jnp.int32)
        return carry

    lax.fori_loop(0, ZSTRIPE // 16, fill_zeros, 0)

    pltpu.sync_copy(zeros_v, hist_sh.at[pl.ds(s * ZSTRIPE, ZSTRIPE)])

    cp_e.wait()
    cp_o.wait()
    plsc.subcore_barrier()
    pltpu.sync_copy(ones_v, hist_sh.at[idx_v], add=True)
    plsc.subcore_barrier()

    @pl.when(c == 0)
    def _():
        pltpu.sync_copy(hist_sh.at[pl.ds(s * ZSTRIPE, ZSTRIPE)],
                        p0_ref.at[pl.ds(s * ZSTRIPE, ZSTRIPE)])

    @pl.when(c == 1)
    def _():
        pltpu.sync_copy(hist_sh.at[pl.ds(s * ZSTRIPE, ZSTRIPE)],
                        p1_ref.at[pl.ds(s * ZSTRIPE, ZSTRIPE)])


_hist_kernel = pl.kernel(
    _hist_body,
    out_type=(jax.ShapeDtypeStruct((HIST_PAD,), jnp.int32),
              jax.ShapeDtypeStruct((HIST_PAD,), jnp.int32)),
    mesh=_mesh,
    scratch_types=[
        pltpu.VMEM((EPW,), jnp.int32),
        pltpu.VMEM((EPW,), jnp.int32),
        pltpu.VMEM((ZSTRIPE,), jnp.int32),
        pltpu.VMEM_SHARED((HIST_PAD,), jnp.int32),
        pltpu.SemaphoreType.DMA,
        pltpu.SemaphoreType.DMA,
    ],
)


_BLK = 2048
_GRID = (N_NODES + _BLK - 1) // _BLK
_SUB = _BLK // 128


def _lookup_body(p0_ref, p1_ref, x_ref, emb_ref, out_ref):
    out_ref[:, 0:D_FEAT] = x_ref[...]
    emb = emb_ref[...].astype(jnp.bfloat16)
    deg = jnp.minimum(p0_ref[...] + p1_ref[...],
                      jnp.int32(IN_CHANNELS - 1))
    iota_k = lax.broadcasted_iota(jnp.int32, (IN_CHANNELS, 128), 0)
    for r in range(_SUB):
        deg_row = deg[r:r + 1, :]
        pt = (jnp.broadcast_to(deg_row, (IN_CHANNELS, 128)) == iota_k)
        pt = pt.astype(jnp.bfloat16)
        e = lax.dot_general(pt, emb, (((0,), (0,)), ((), ())),
                            preferred_element_type=jnp.float32)
        out_ref[pl.ds(r * 128, 128), pl.ds(D_FEAT, OUT_CHANNELS)] = e


_lookup_kernel = pl.pallas_call(
    _lookup_body,
    grid=(_GRID,),
    in_specs=[
        pl.BlockSpec((_SUB, 128), lambda i: (i, 0)),
        pl.BlockSpec((_SUB, 128), lambda i: (i, 0)),
        pl.BlockSpec((_BLK, D_FEAT), lambda i: (i, 0)),
        pl.BlockSpec((IN_CHANNELS, OUT_CHANNELS), lambda i: (0, 0)),
    ],
    out_specs=pl.BlockSpec((_BLK, D_FEAT + OUT_CHANNELS), lambda i: (i, 0)),
    out_shape=jax.ShapeDtypeStruct((N_NODES, D_FEAT + OUT_CHANNELS),
                                   jnp.float32),
)


def kernel(x, edge_index, emb_table):
    edge_flat = edge_index.astype(jnp.int32).reshape(2 * N_EDGES)
    ones = jnp.full((EPW,), 1, jnp.int32)
    p0, p1 = _hist_kernel(edge_flat, ones)
    p0v = p0.reshape(HIST_PAD // 128, 128)
    p1v = p1.reshape(HIST_PAD // 128, 128)
    return _lookup_kernel(p0v, p1v, x, emb_table)
